# baseline (speedup 1.0000x reference)
import functools

import jax
import jax.numpy as jnp
from jax import lax
from jax.experimental import pallas as pl
from jax.experimental.pallas import tpu as pltpu
from jax.experimental.pallas import tpu_sc as plsc

D = 32
NW = 32
CHUNK = 1024
NBUF = 3


def _make_gather(b_total: int, n_rows: int):
    b_per_w = b_total // NW
    n_chunks = b_per_w // CHUNK
    mesh = plsc.VectorSubcoreMesh(core_axis_name="c", subcore_axis_name="s")

    @functools.partial(
        pl.kernel,
        mesh=mesh,
        compiler_params=pltpu.CompilerParams(use_tc_tiling_on_sc=False),
        out_type=jax.ShapeDtypeStruct((b_total, D), jnp.float32),
        scratch_types=[
            pltpu.VMEM((b_per_w,), jnp.int32),
            pltpu.VMEM((NBUF, CHUNK, D), jnp.float32),
            pltpu.SemaphoreType.DMA((NBUF,)),
            pltpu.SemaphoreType.DMA((NBUF,)),
        ],
    )
    def gather(idx_hbm, table_hbm, out_hbm, idx_v, rows_v, gsem, ssem):
        wid = lax.axis_index("s") * 2 + lax.axis_index("c")
        base = wid * b_per_w
        pltpu.sync_copy(idx_hbm.at[pl.ds(base, b_per_w)], idx_v)

        def start_gather(i, b):
            return pltpu.async_copy(
                table_hbm.at[idx_v.at[pl.ds(i * CHUNK, CHUNK)]],
                rows_v.at[b],
                gsem.at[b],
            )

        gcp = [None] * n_chunks
        scp = [None] * n_chunks
        for i in range(min(NBUF, n_chunks)):
            gcp[i] = start_gather(i, i)
        for i in range(n_chunks):
            b = i % NBUF
            gcp[i].wait()
            scp[i] = pltpu.async_copy(
                rows_v.at[b],
                out_hbm.at[pl.ds(base + i * CHUNK, CHUNK)],
                ssem.at[b],
            )
            nxt = i + NBUF
            if nxt < n_chunks:
                scp[i].wait()
                gcp[nxt] = start_gather(nxt, b)
        for i in range(max(0, n_chunks - NBUF), n_chunks):
            scp[i].wait()

    return gather


_TK = 16


def _transpose_to_rm(tab_t):
    d, n = tab_t.shape
    n_out = n * d // 128
    grid = (n + 128 * _TK - 1) // (128 * _TK)

    def body(x_ref, o_ref):
        xt = x_ref[...].T
        x3 = xt.reshape(_TK * d, 128 // d, d)
        o_ref[...] = jnp.concatenate(
            [x3[:, m, :] for m in range(128 // d)], axis=1
        )

    return pl.pallas_call(
        body,
        grid=(grid,),
        in_specs=[pl.BlockSpec((d, 128 * _TK), lambda g: (0, g))],
        out_specs=pl.BlockSpec((_TK * d, 128), lambda g: (g, 0)),
        out_shape=jax.ShapeDtypeStruct((n_out, 128), jnp.float32),
    )(tab_t)


def kernel(token_ids, embedding_matrix):
    nb, s = token_ids.shape
    n_rows, d = embedding_matrix.shape
    flat = token_ids.reshape(nb * s).astype(jnp.int32)
    tab_rm = _transpose_to_rm(embedding_matrix.T)
    tab = tab_rm.reshape(n_rows, d)
    out = _make_gather(nb * s, n_rows)(flat, tab)
    return out.reshape(nb, s, d)


# speedup vs baseline: 1.1359x; 1.1359x over previous
import functools

import jax
import jax.numpy as jnp
from jax import lax
from jax.experimental import pallas as pl
from jax.experimental.pallas import tpu as pltpu
from jax.experimental.pallas import tpu_sc as plsc

D = 32
NW = 32
CHUNK = 1024
NBUF = 3


def _make_gather(b_total: int, n_rows: int):
    b_per_w = b_total // NW
    n_chunks = b_per_w // CHUNK
    mesh = plsc.VectorSubcoreMesh(core_axis_name="c", subcore_axis_name="s")

    @functools.partial(
        pl.kernel,
        mesh=mesh,
        compiler_params=pltpu.CompilerParams(use_tc_tiling_on_sc=False),
        out_type=jax.ShapeDtypeStruct((b_total, D), jnp.float32),
        scratch_types=[
            pltpu.VMEM((b_per_w,), jnp.int32),
            pltpu.VMEM((NBUF, CHUNK, D), jnp.float32),
            pltpu.SemaphoreType.DMA((NBUF,)),
            pltpu.SemaphoreType.DMA((NBUF,)),
        ],
    )
    def gather(idx_hbm, table_hbm, out_hbm, idx_v, rows_v, gsem, ssem):
        wid = lax.axis_index("s") * 2 + lax.axis_index("c")
        base = wid * b_per_w
        pltpu.sync_copy(idx_hbm.at[pl.ds(base, b_per_w)], idx_v)

        def start_gather(i, b):
            return pltpu.async_copy(
                table_hbm.at[idx_v.at[pl.ds(i * CHUNK, CHUNK)]],
                rows_v.at[b],
                gsem.at[b],
            )

        gcp = [None] * n_chunks
        scp = [None] * n_chunks
        for i in range(min(NBUF, n_chunks)):
            gcp[i] = start_gather(i, i)
        for i in range(n_chunks):
            b = i % NBUF
            gcp[i].wait()
            scp[i] = pltpu.async_copy(
                rows_v.at[b],
                out_hbm.at[pl.ds(base + i * CHUNK, CHUNK)],
                ssem.at[b],
            )
            nxt = i + NBUF
            if nxt < n_chunks:
                scp[i].wait()
                gcp[nxt] = start_gather(nxt, b)
        for i in range(max(0, n_chunks - NBUF), n_chunks):
            scp[i].wait()

    return gather


_TK = 63


def _transpose_to_rm(tab_t):
    d, n = tab_t.shape
    n_out = n * d // 128
    grid = (n + 128 * _TK - 1) // (128 * _TK)

    def body(x_ref, o_ref):
        xt = x_ref[...].T
        x3 = xt.reshape(_TK * d, 128 // d, d)
        o_ref[...] = jnp.concatenate(
            [x3[:, m, :] for m in range(128 // d)], axis=1
        )

    return pl.pallas_call(
        body,
        grid=(grid,),
        in_specs=[pl.BlockSpec((d, 128 * _TK), lambda g: (0, g))],
        out_specs=pl.BlockSpec((_TK * d, 128), lambda g: (g, 0)),
        out_shape=jax.ShapeDtypeStruct((n_out, 128), jnp.float32),
    )(tab_t)


def kernel(token_ids, embedding_matrix):
    nb, s = token_ids.shape
    n_rows, d = embedding_matrix.shape
    flat = token_ids.reshape(nb * s).astype(jnp.int32)
    tab_rm = _transpose_to_rm(embedding_matrix.T)
    tab = tab_rm.reshape(n_rows, d)
    out = _make_gather(nb * s, n_rows)(flat, tab)
    return out.reshape(nb, s, d)


# speedup vs baseline: 1.1505x; 1.0129x over previous
import functools

import jax
import jax.numpy as jnp
from jax import lax
from jax.experimental import pallas as pl
from jax.experimental.pallas import tpu as pltpu
from jax.experimental.pallas import tpu_sc as plsc

D = 32
NW = 32
CHUNK = 1024
NBUF = 3


def _make_gather(b_total: int, n_rows: int):
    b_per_w = b_total // NW
    n_chunks = b_per_w // CHUNK
    mesh = plsc.VectorSubcoreMesh(core_axis_name="c", subcore_axis_name="s")

    @functools.partial(
        pl.kernel,
        mesh=mesh,
        compiler_params=pltpu.CompilerParams(use_tc_tiling_on_sc=False),
        out_type=jax.ShapeDtypeStruct((b_total, D), jnp.float32),
        scratch_types=[
            pltpu.VMEM((b_per_w,), jnp.int32),
            pltpu.VMEM((NBUF, CHUNK, D), jnp.float32),
            pltpu.SemaphoreType.DMA((NBUF,)),
            pltpu.SemaphoreType.DMA((NBUF,)),
        ],
    )
    def gather(idx_hbm, table_hbm, out_hbm, idx_v, rows_v, gsem, ssem):
        wid = lax.axis_index("s") * 2 + lax.axis_index("c")
        base = wid * b_per_w
        pltpu.sync_copy(idx_hbm.at[pl.ds(base, b_per_w)], idx_v)

        def start_gather(i, b):
            return pltpu.async_copy(
                table_hbm.at[idx_v.at[pl.ds(i * CHUNK, CHUNK)]],
                rows_v.at[b],
                gsem.at[b],
            )

        gcp = [None] * n_chunks
        scp = [None] * n_chunks
        for i in range(min(NBUF, n_chunks)):
            gcp[i] = start_gather(i, i)
        for i in range(n_chunks):
            b = i % NBUF
            gcp[i].wait()
            scp[i] = pltpu.async_copy(
                rows_v.at[b],
                out_hbm.at[pl.ds(base + i * CHUNK, CHUNK)],
                ssem.at[b],
            )
            nxt = i + NBUF
            if nxt < n_chunks:
                scp[i].wait()
                gcp[nxt] = start_gather(nxt, b)
        for i in range(max(0, n_chunks - NBUF), n_chunks):
            scp[i].wait()

    return gather


_TK = 126


def _transpose_to_rm(tab_t):
    d, n = tab_t.shape
    n_out = n * d // 128
    grid = (n + 128 * _TK - 1) // (128 * _TK)

    def body(x_ref, o_ref):
        xt = x_ref[...].T
        x3 = xt.reshape(_TK * d, 128 // d, d)
        o_ref[...] = jnp.concatenate(
            [x3[:, m, :] for m in range(128 // d)], axis=1
        )

    return pl.pallas_call(
        body,
        grid=(grid,),
        in_specs=[pl.BlockSpec((d, 128 * _TK), lambda g: (0, g))],
        out_specs=pl.BlockSpec((_TK * d, 128), lambda g: (g, 0)),
        out_shape=jax.ShapeDtypeStruct((n_out, 128), jnp.float32),
    )(tab_t)


def kernel(token_ids, embedding_matrix):
    nb, s = token_ids.shape
    n_rows, d = embedding_matrix.shape
    flat = token_ids.reshape(nb * s).astype(jnp.int32)
    tab_rm = _transpose_to_rm(embedding_matrix.T)
    tab = tab_rm.reshape(n_rows, d)
    out = _make_gather(nb * s, n_rows)(flat, tab)
    return out.reshape(nb, s, d)
